# baseline (speedup 1.0000x reference)
import functools

import jax
import jax.numpy as jnp
from jax import lax
from jax.experimental import pallas as pl
from jax.experimental.pallas import tpu as pltpu
from jax.experimental.pallas import tpu_sc as plsc
from jax.experimental.shard_map import shard_map
from jax.sharding import PartitionSpec as P

_N_E = 8192
_E_DIM = 256
_BETA = 0.25
_T = 256
_SLAB1 = 2736
_SLAB2 = 5472


def _vq_kernel(zt_ref, se_ref, emb_ref, lsum_ref, counts_ref, enc_ref,
               idx_ref):
    i = pl.program_id(0)

    zt = zt_ref[...]
    sz = jnp.sum(zt * zt, axis=1, keepdims=True)
    se = se_ref[...]
    emb = emb_ref[...]

    s2 = jax.lax.dot_general(-2.0 * zt, emb, (((1,), (1,)), ((), ())),
                             preferred_element_type=jnp.float32)
    d = (sz + se) + s2

    io = jax.lax.broadcasted_iota(
        jnp.int32, (1, _N_E), 1).astype(jnp.float32)
    inf = jnp.float32(jnp.inf)
    big = jnp.float32(_N_E)
    b0lo, b0hi = 2688, 2816
    b1lo, b1hi = 5376, 5504
    db0, db1 = d[:, b0lo:b0hi], d[:, b1lo:b1hi]
    iob0, iob1 = io[:, b0lo:b0hi], io[:, b1lo:b1hi]
    mk0, mk1 = iob0 < _SLAB1, iob1 < _SLAB2

    def _rmin(x):
        return jnp.min(x, axis=1, keepdims=True)

    m0 = jnp.minimum(_rmin(d[:, :b0lo]), _rmin(jnp.where(mk0, db0, inf)))
    m1 = jnp.minimum(
        jnp.minimum(_rmin(jnp.where(mk0, inf, db0)), _rmin(d[:, b0hi:b1lo])),
        _rmin(jnp.where(mk1, db1, inf)))
    m2 = jnp.minimum(_rmin(jnp.where(mk1, inf, db1)), _rmin(d[:, b1hi:]))

    def _ridx(x, m, ios):
        return jnp.min(jnp.where(x == m, ios, big), axis=1, keepdims=True)

    i0 = jnp.minimum(
        _ridx(d[:, :b0lo], m0, io[:, :b0lo]),
        _ridx(jnp.where(mk0, db0, inf), m0, iob0))
    i1 = jnp.minimum(
        jnp.minimum(_ridx(jnp.where(mk0, inf, db0), m1, iob0),
                    _ridx(d[:, b0hi:b1lo], m1, io[:, b0hi:b1lo])),
        _ridx(jnp.where(mk1, db1, inf), m1, iob1))
    i2 = jnp.minimum(_ridx(jnp.where(mk1, inf, db1), m2, iob1),
                     _ridx(d[:, b1hi:], m2, io[:, b1hi:]))

    q0 = m0.astype(jnp.bfloat16).astype(jnp.float32)
    t1 = m1 < q0
    v1 = jnp.where(t1, m1, q0)
    q1 = v1.astype(jnp.bfloat16).astype(jnp.float32)
    t2 = m2 < q1
    v_sel = jnp.where(t2, m2, jnp.where(t1, m1, m0))
    idxf = jnp.where(t2, i2, jnp.where(t1, i1, i0))
    idx_ref[...] = idxf.astype(jnp.int32)

    onehot = (io == idxf).astype(jnp.float32)
    enc_ref[...] = onehot

    @pl.when(i == 0)
    def _init():
        counts_ref[...] = jnp.zeros_like(counts_ref)
        lsum_ref[...] = jnp.zeros_like(lsum_ref)

    counts_ref[...] += jnp.sum(onehot, axis=0, keepdims=True)
    lsum_ref[...] += jnp.sum(v_sel, keepdims=True)


def _fin_kernel(n_tok, lsum_ref, counts_ref, loss_ref, perp_ref):
    loss_ref[...] = (1.0 + _BETA) * lsum_ref[...] / (n_tok * _E_DIM)
    e_mean = counts_ref[...] / n_tok
    ent = jnp.sum(e_mean * jnp.log(e_mean + 1e-10), keepdims=True)
    perp_ref[...] = jnp.exp(-ent)


def _make_sc_gather(n_tok):
    info = plsc.get_sparse_core_info()
    nw = info.num_cores * info.num_subcores
    b_per_w = n_tok // nw
    chunk = min(b_per_w, 256)
    n_chunks = b_per_w // chunk
    mesh = plsc.VectorSubcoreMesh(core_axis_name="c", subcore_axis_name="s")

    @functools.partial(
        pl.kernel, mesh=mesh,
        out_type=jax.ShapeDtypeStruct((n_tok, _E_DIM), jnp.float32),
        scratch_types=[
            pltpu.VMEM((chunk,), jnp.int32),
            pltpu.VMEM((chunk, _E_DIM), jnp.float32),
            pltpu.SemaphoreType.DMA,
        ],
    )
    def gather_k(table_hbm, idx_hbm, out_hbm, idx_v, rows_v, sem):
        wid = lax.axis_index("s") * info.num_cores + lax.axis_index("c")
        for c in range(n_chunks):
            base = wid * b_per_w + c * chunk
            pltpu.sync_copy(idx_hbm.at[pl.ds(base, chunk)], idx_v)
            pltpu.async_copy(table_hbm.at[idx_v], rows_v, sem).wait()
            pltpu.sync_copy(rows_v, out_hbm.at[pl.ds(base, chunk)])

    return gather_k


def kernel(z, emb):
    B, C, H, W = z.shape
    n_tok_total = B * H * W
    ndev = min(2, jax.device_count())
    if B % ndev:
        ndev = 1
    mesh = jax.make_mesh((ndev,), ("x",),
                         axis_types=(jax.sharding.AxisType.Auto,))

    def _shard_fn(z_sh, emb):
        b_sh = z_sh.shape[0]
        zp = jnp.transpose(z_sh, (0, 2, 3, 1))
        z_flat = zp.reshape(-1, _E_DIM)
        n_tok = z_flat.shape[0]
        se = jnp.sum(emb ** 2, axis=1)[None, :]

        lsum, counts, enc, idx = pl.pallas_call(
            _vq_kernel,
            grid=(n_tok // _T,),
            in_specs=[
                pl.BlockSpec((_T, _E_DIM), lambda i: (i, 0)),
                pl.BlockSpec((1, _N_E), lambda i: (0, 0)),
                pl.BlockSpec((_N_E, _E_DIM), lambda i: (0, 0)),
            ],
            out_specs=[
                pl.BlockSpec((1, 1), lambda i: (0, 0)),
                pl.BlockSpec((1, _N_E), lambda i: (0, 0)),
                pl.BlockSpec((_T, _N_E), lambda i: (i, 0)),
                pl.BlockSpec((_T, 1), lambda i: (i, 0)),
            ],
            out_shape=[
                jax.ShapeDtypeStruct((1, 1), jnp.float32),
                jax.ShapeDtypeStruct((1, _N_E), jnp.float32),
                jax.ShapeDtypeStruct((n_tok, _N_E), jnp.float32),
                jax.ShapeDtypeStruct((n_tok, 1), jnp.int32),
            ],
        )(z_flat, se, emb)

        lsum = jax.lax.psum(lsum, "x")
        counts = jax.lax.psum(counts, "x")
        loss, perp = pl.pallas_call(
            functools.partial(_fin_kernel, n_tok_total),
            out_shape=[
                jax.ShapeDtypeStruct((1, 1), jnp.float32),
                jax.ShapeDtypeStruct((1, 1), jnp.float32),
            ],
        )(lsum, counts)

        zq_flat = _make_sc_gather(n_tok)(emb, idx.reshape(-1))
        z_q = jnp.transpose(zq_flat.reshape(b_sh, H, W, C), (0, 3, 1, 2))
        return loss, z_q, perp, enc, idx

    from jax.sharding import NamedSharding
    z = jax.lax.with_sharding_constraint(z, NamedSharding(mesh, P("x")))
    emb = jax.lax.with_sharding_constraint(emb, NamedSharding(mesh, P()))
    loss, z_q, perp, enc, idx = shard_map(
        _shard_fn, mesh=mesh,
        in_specs=(P("x"), P()),
        out_specs=(P(), P("x"), P(), P("x"), P("x")),
        check_rep=False,
    )(z, emb)
    return (loss[0, 0], z_q, perp[0, 0], enc, idx)


# speedup vs baseline: 1.2466x; 1.2466x over previous
import functools

import jax
import jax.numpy as jnp
from jax import lax
from jax.experimental import pallas as pl
from jax.experimental.pallas import tpu as pltpu
from jax.experimental.pallas import tpu_sc as plsc

_N_E = 8192
_E_DIM = 256
_BETA = 0.25
_T = 256
_SLAB1 = 2736
_SLAB2 = 5472


def _vq_kernel(zt_ref, se_ref, emb_ref, loss_ref, perp_ref, enc_ref,
               idx_ref, counts_sc, loss_sc):
    i = pl.program_id(0)
    n_tok = pl.num_programs(0) * _T

    zt = zt_ref[...]
    sz = jnp.sum(zt * zt, axis=1, keepdims=True)
    se = se_ref[...]
    emb = emb_ref[...]

    s2 = jax.lax.dot_general(-2.0 * zt, emb, (((1,), (1,)), ((), ())),
                             preferred_element_type=jnp.float32)
    d = (sz + se) + s2

    io = jax.lax.broadcasted_iota(
        jnp.int32, (1, _N_E), 1).astype(jnp.float32)
    inf = jnp.float32(jnp.inf)
    big = jnp.float32(_N_E)
    b0lo, b0hi = 2688, 2816
    b1lo, b1hi = 5376, 5504
    db0, db1 = d[:, b0lo:b0hi], d[:, b1lo:b1hi]
    iob0, iob1 = io[:, b0lo:b0hi], io[:, b1lo:b1hi]
    mk0, mk1 = iob0 < _SLAB1, iob1 < _SLAB2

    def _slab_scan(cols):
        (d0, io0) = cols[0]
        acc_v = d0
        acc_i = jnp.broadcast_to(io0, d0.shape)
        for dc, ioc in cols[1:]:
            take = dc < acc_v
            acc_i = jnp.where(take, jnp.broadcast_to(ioc, dc.shape), acc_i)
            acc_v = jnp.minimum(acc_v, dc)
        m = jnp.min(acc_v, axis=1, keepdims=True)
        idx = jnp.min(jnp.where(acc_v == m, acc_i, big), axis=1, keepdims=True)
        return m, idx

    def _cols(lo, hi):
        return [(d[:, c:c + 128], io[:, c:c + 128]) for c in range(lo, hi, 128)]

    m0, i0 = _slab_scan(_cols(0, b0lo) + [(jnp.where(mk0, db0, inf), iob0)])
    m1, i1 = _slab_scan([(jnp.where(mk0, inf, db0), iob0)]
                        + _cols(b0hi, b1lo)
                        + [(jnp.where(mk1, db1, inf), iob1)])
    m2, i2 = _slab_scan([(jnp.where(mk1, inf, db1), iob1)]
                        + _cols(b1hi, _N_E))

    q0 = m0.astype(jnp.bfloat16).astype(jnp.float32)
    t1 = m1 < q0
    v1 = jnp.where(t1, m1, q0)
    q1 = v1.astype(jnp.bfloat16).astype(jnp.float32)
    t2 = m2 < q1
    v_sel = jnp.where(t2, m2, jnp.where(t1, m1, m0))
    idxf = jnp.where(t2, i2, jnp.where(t1, i1, i0))
    idx_ref[...] = idxf.astype(jnp.int32)

    onehot = (io == idxf).astype(jnp.float32)
    enc_ref[...] = onehot

    @pl.when(i == 0)
    def _init():
        counts_sc[...] = jnp.zeros_like(counts_sc)
        loss_sc[...] = jnp.zeros_like(loss_sc)

    counts_sc[...] += jnp.sum(onehot, axis=0, keepdims=True)
    loss_sc[...] += jnp.sum(v_sel, keepdims=True)

    @pl.when(i == pl.num_programs(0) - 1)
    def _finalize():
        loss_ref[...] = (1.0 + _BETA) * loss_sc[...] / (n_tok * _E_DIM)
        e_mean = counts_sc[...] / n_tok
        ent = jnp.sum(e_mean * jnp.log(e_mean + 1e-10), keepdims=True)
        perp_ref[...] = jnp.exp(-ent)


def _make_sc_gather(n_tok):
    info = plsc.get_sparse_core_info()
    nw = info.num_cores * info.num_subcores
    b_per_w = n_tok // nw
    chunk = min(b_per_w, 256)
    n_chunks = b_per_w // chunk
    mesh = plsc.VectorSubcoreMesh(core_axis_name="c", subcore_axis_name="s")

    @functools.partial(
        pl.kernel, mesh=mesh,
        out_type=jax.ShapeDtypeStruct((n_tok, _E_DIM), jnp.float32),
        scratch_types=[
            pltpu.VMEM((chunk,), jnp.int32),
            pltpu.VMEM((chunk, _E_DIM), jnp.float32),
            pltpu.SemaphoreType.DMA,
        ],
    )
    def gather_k(table_hbm, idx_hbm, out_hbm, idx_v, rows_v, sem):
        wid = lax.axis_index("s") * info.num_cores + lax.axis_index("c")
        for c in range(n_chunks):
            base = wid * b_per_w + c * chunk
            pltpu.sync_copy(idx_hbm.at[pl.ds(base, chunk)], idx_v)
            pltpu.async_copy(table_hbm.at[idx_v], rows_v, sem).wait()
            pltpu.sync_copy(rows_v, out_hbm.at[pl.ds(base, chunk)])

    return gather_k


def kernel(z, emb):
    B, C, H, W = z.shape
    zp = jnp.transpose(z, (0, 2, 3, 1))
    z_flat = zp.reshape(-1, _E_DIM)
    n_tok = z_flat.shape[0]
    se = jnp.sum(emb ** 2, axis=1)[None, :]

    grid = (n_tok // _T,)
    loss, perp, enc, idx = pl.pallas_call(
        _vq_kernel,
        grid=grid,
        in_specs=[
            pl.BlockSpec((_T, _E_DIM), lambda i: (i, 0)),
            pl.BlockSpec((1, _N_E), lambda i: (0, 0)),
            pl.BlockSpec((_N_E, _E_DIM), lambda i: (0, 0)),
        ],
        out_specs=[
            pl.BlockSpec((1, 1), lambda i: (0, 0)),
            pl.BlockSpec((1, 1), lambda i: (0, 0)),
            pl.BlockSpec((_T, _N_E), lambda i: (i, 0)),
            pl.BlockSpec((_T, 1), lambda i: (i, 0)),
        ],
        out_shape=[
            jax.ShapeDtypeStruct((1, 1), jnp.float32),
            jax.ShapeDtypeStruct((1, 1), jnp.float32),
            jax.ShapeDtypeStruct((n_tok, _N_E), jnp.float32),
            jax.ShapeDtypeStruct((n_tok, 1), jnp.int32),
        ],
        scratch_shapes=[
            pltpu.VMEM((1, _N_E), jnp.float32),
            pltpu.VMEM((1, 1), jnp.float32),
        ],
    )(z_flat, se, emb)

    zq_flat = _make_sc_gather(n_tok)(emb, idx.reshape(-1))
    z_q = jnp.transpose(zq_flat.reshape(B, H, W, C), (0, 3, 1, 2))
    return (loss[0, 0], z_q, perp[0, 0], enc, idx)
